# baseline (speedup 1.0000x reference)
import functools

import jax
import jax.numpy as jnp
from jax.experimental import pallas as pl
from jax.experimental.pallas import tpu as pltpu

_BLOCK = 2048


def _nn_block_kernel(nb, blk, x2_ref, a2_ref, x1t_ref, b2_ref,
                     corr_ref, idx12_ref, idx21_ref, rv_ref, ri_ref):
    i = pl.program_id(0)
    n2 = x2_ref.shape[0]

    ab = jnp.dot(x2_ref[...], x1t_ref[...])
    d2 = a2_ref[...] + b2_ref[0] - 2.0 * ab
    d = jnp.sqrt(jnp.maximum(d2, 0.0))
    corr_ref[...] = d

    cmin = jnp.min(d, axis=0, keepdims=True)
    rows = jax.lax.broadcasted_iota(jnp.int32, (n2, blk), 0)
    idx12_ref[0] = jnp.min(jnp.where(d == cmin, rows, n2),
                           axis=0, keepdims=True)

    tmin = jnp.min(d, axis=1, keepdims=True)
    cols = jax.lax.broadcasted_iota(jnp.int32, (n2, blk), 1)
    tloc = jnp.min(jnp.where(d == tmin, cols, blk),
                   axis=1, keepdims=True)
    tidx = tloc + i * blk

    @pl.when(i == 0)
    def _init():
        rv_ref[...] = jnp.full(rv_ref.shape, jnp.inf, rv_ref.dtype)
        ri_ref[...] = jnp.zeros(ri_ref.shape, ri_ref.dtype)

    better = tmin < rv_ref[...]
    rv_ref[...] = jnp.where(better, tmin, rv_ref[...])
    ri_ref[...] = jnp.where(better, tidx, ri_ref[...])

    @pl.when(i == nb - 1)
    def _fin():
        idx21_ref[...] = ri_ref[...]


@jax.jit
def kernel(x1, x2):
    n1, feat = x1.shape
    n2 = x2.shape[0]
    blk = _BLOCK
    nb = -(-n1 // blk)
    n1p = nb * blk

    a2 = jnp.sum(x2 * x2, axis=-1, keepdims=True)
    b2 = jnp.sum(x1 * x1, axis=-1, keepdims=True)
    b2p = jnp.pad(b2.T, ((0, 0), (0, n1p - n1)),
                  constant_values=jnp.inf).reshape(nb, 1, blk)
    x1tp = jnp.pad(x1.T, ((0, 0), (0, n1p - n1)))

    corr, idx12p, idx21 = pl.pallas_call(
        functools.partial(_nn_block_kernel, nb, blk),
        grid=(nb,),
        in_specs=[
            pl.BlockSpec((n2, feat), lambda i: (0, 0)),
            pl.BlockSpec((n2, 1), lambda i: (0, 0)),
            pl.BlockSpec((feat, blk), lambda i: (0, i)),
            pl.BlockSpec((1, 1, blk), lambda i: (i, 0, 0)),
        ],
        out_specs=[
            pl.BlockSpec((n2, blk), lambda i: (0, i)),
            pl.BlockSpec((1, 1, blk), lambda i: (i, 0, 0)),
            pl.BlockSpec((n2, 1), lambda i: (0, 0)),
        ],
        out_shape=[
            jax.ShapeDtypeStruct((n2, n1), jnp.float32),
            jax.ShapeDtypeStruct((nb, 1, blk), jnp.int32),
            jax.ShapeDtypeStruct((n2, 1), jnp.int32),
        ],
        scratch_shapes=[
            pltpu.VMEM((n2, 1), jnp.float32),
            pltpu.VMEM((n2, 1), jnp.int32),
        ],
    )(x2, a2, x1tp, b2p)

    return (x1, x2, corr, idx12p.reshape(n1p)[:n1], idx21.reshape(n2))


# speedup vs baseline: 1.4163x; 1.4163x over previous
import functools

import jax
import jax.numpy as jnp
from jax.experimental import pallas as pl
from jax.experimental.pallas import tpu as pltpu

_BLOCK = 2048


def _nn_block_kernel(nb, blk, x2_ref, a2_ref, x1_ref, b2_ref,
                     corr_ref, idx12_ref, idx21_ref, rv_ref, ri_ref):
    i = pl.program_id(0)
    n2 = x2_ref.shape[0]

    abt = jax.lax.dot_general(x1_ref[...], x2_ref[...],
                              (((1,), (1,)), ((), ())))
    d2 = b2_ref[...] + a2_ref[...] - 2.0 * abt
    d = jnp.sqrt(jnp.maximum(d2, 0.0))
    corr_ref[...] = d

    rmin = jnp.min(d, axis=1, keepdims=True)
    cols = jax.lax.broadcasted_iota(jnp.int32, (blk, n2), 1)
    idx12_ref[0] = jnp.min(jnp.where(d == rmin, cols, n2),
                           axis=1, keepdims=True)

    cmin = jnp.min(d, axis=0, keepdims=True)
    rows = jax.lax.broadcasted_iota(jnp.int32, (blk, n2), 0)
    tloc = jnp.min(jnp.where(d == cmin, rows, blk),
                   axis=0, keepdims=True)
    tidx = tloc + i * blk

    @pl.when(i == 0)
    def _init():
        rv_ref[...] = jnp.full(rv_ref.shape, jnp.inf, rv_ref.dtype)
        ri_ref[...] = jnp.zeros(ri_ref.shape, ri_ref.dtype)

    better = cmin < rv_ref[...]
    rv_ref[...] = jnp.where(better, cmin, rv_ref[...])
    ri_ref[...] = jnp.where(better, tidx, ri_ref[...])

    @pl.when(i == nb - 1)
    def _fin():
        idx21_ref[...] = ri_ref[...]


@jax.jit
def kernel(x1, x2):
    n1, feat = x1.shape
    n2 = x2.shape[0]
    blk = _BLOCK
    nb = -(-n1 // blk)
    n1p = nb * blk

    a2 = jnp.sum(x2 * x2, axis=-1, keepdims=True)
    b2 = jnp.sum(x1 * x1, axis=-1, keepdims=True)
    b2p = jnp.pad(b2, ((0, n1p - n1), (0, 0)), constant_values=jnp.inf)
    x1p = jnp.pad(x1, ((0, n1p - n1), (0, 0)))

    corrt, idx12p, idx21 = pl.pallas_call(
        functools.partial(_nn_block_kernel, nb, blk),
        grid=(nb,),
        in_specs=[
            pl.BlockSpec((n2, feat), lambda i: (0, 0)),
            pl.BlockSpec((1, n2), lambda i: (0, 0)),
            pl.BlockSpec((blk, feat), lambda i: (i, 0)),
            pl.BlockSpec((blk, 1), lambda i: (i, 0)),
        ],
        out_specs=[
            pl.BlockSpec((blk, n2), lambda i: (i, 0)),
            pl.BlockSpec((1, blk, 1), lambda i: (i, 0, 0)),
            pl.BlockSpec((1, n2), lambda i: (0, 0)),
        ],
        out_shape=[
            jax.ShapeDtypeStruct((n1, n2), jnp.float32),
            jax.ShapeDtypeStruct((nb, blk, 1), jnp.int32),
            jax.ShapeDtypeStruct((1, n2), jnp.int32),
        ],
        scratch_shapes=[
            pltpu.VMEM((1, n2), jnp.float32),
            pltpu.VMEM((1, n2), jnp.int32),
        ],
    )(x2, a2.T, x1p, b2p)

    return (x1, x2, corrt.T, idx12p.reshape(n1p)[:n1], idx21.reshape(n2))
